# baseline (speedup 1.0000x reference)
import functools

import jax
import jax.numpy as jnp
from jax import lax
from jax.experimental import pallas as pl
from jax.experimental.pallas import tpu as pltpu
from jax.experimental.pallas import tpu_sc as plsc

_D = 128
_NW = 32
_CH = 128
_NBUF = 4

_MESH = plsc.VectorSubcoreMesh(core_axis_name="c", subcore_axis_name="s")


def _phase(table, idx1, out, dummy, idx_v, bufs, gsems, wsems, wid, nrpw):
    nchunks = nrpw // _CH
    base = wid * nrpw

    pltpu.sync_copy(idx1.at[pl.ds(base, nrpw)], idx_v.at[pl.ds(0, nrpw)])

    def gather(c, b):
        pltpu.async_copy(table.at[idx_v.at[pl.ds(c * _CH, _CH)]],
                         bufs[b], gsems[b])

    for b in range(_NBUF - 1):
        gather(b, b)

    def proc(c, b):
        bn = (b + _NBUF - 1) % _NBUF
        @pl.when(c > 0)
        def _():
            pltpu.make_async_copy(bufs[bn], dummy, wsems[bn]).wait()

        @pl.when(c + _NBUF - 1 < nchunks)
        def _():
            gather(c + _NBUF - 1, bn)

        pltpu.make_async_copy(dummy, bufs[b], gsems[b]).wait()
        pltpu.async_copy(bufs[b], out.at[pl.ds(base + c * _CH, _CH)],
                         wsems[b])

    def body(jj, carry):
        for b in range(_NBUF):
            proc(_NBUF * jj + b, b)
        return carry

    lax.fori_loop(0, nchunks // _NBUF, body, 0)
    pltpu.make_async_copy(bufs[_NBUF - 1], dummy, wsems[_NBUF - 1]).wait()


def _make_small_kernel(B, St, Sc, Vt, Vc):
    nmax = B * max(St, Sc) // _NW

    @functools.partial(
        pl.kernel,
        mesh=_MESH,
        out_type=(
            jax.ShapeDtypeStruct((B * St, _D), jnp.float32),
            jax.ShapeDtypeStruct((B * Sc, _D), jnp.float32),
        ),
        scratch_types=(
            [pltpu.VMEM((nmax,), jnp.int32)]
            + [pltpu.VMEM((_CH, _D), jnp.float32)] * _NBUF
            + [pltpu.SemaphoreType.DMA] * (2 * _NBUF)
            + [pltpu.VMEM_SHARED((Vt, _D), jnp.float32),
               pltpu.VMEM_SHARED((Vc, _D), jnp.float32)]
        ),
    )
    def ka(type_table, cond_table, t_idx, c_idx, t_out, c_out, idx_v,
           *rest):
        bufs = list(rest[:_NBUF])
        gsems = list(rest[_NBUF:2 * _NBUF])
        wsems = list(rest[2 * _NBUF:3 * _NBUF])
        type_sh, cond_sh = rest[3 * _NBUF:]
        wid = lax.axis_index("s") * _MESH.num_cores + lax.axis_index("c")
        dummy = type_table.at[pl.ds(0, _CH)]
        args = (dummy, idx_v, bufs, gsems, wsems, wid)

        @pl.when(lax.axis_index("s") == 0)
        def _():
            pltpu.sync_copy(type_table, type_sh)
            pltpu.sync_copy(cond_table, cond_sh)

        plsc.subcore_barrier()
        _phase(type_sh, t_idx, t_out, *args, B * St // _NW)
        _phase(cond_sh, c_idx, c_out, *args, B * Sc // _NW)

    return ka


def _make_move_kernel(B, Sm):
    @functools.partial(
        pl.kernel,
        mesh=_MESH,
        out_type=jax.ShapeDtypeStruct((B * Sm, _D), jnp.float32),
        scratch_types=(
            [pltpu.VMEM((B * Sm // _NW,), jnp.int32)]
            + [pltpu.VMEM((_CH, _D), jnp.float32)] * _NBUF
            + [pltpu.SemaphoreType.DMA] * (2 * _NBUF)
        ),
    )
    def kb(move_table, m_idx, m_out, idx_v, *rest):
        bufs = list(rest[:_NBUF])
        gsems = list(rest[_NBUF:2 * _NBUF])
        wsems = list(rest[2 * _NBUF:3 * _NBUF])
        wid = lax.axis_index("s") * _MESH.num_cores + lax.axis_index("c")
        dummy = move_table.at[pl.ds(0, _CH)]
        _phase(move_table, m_idx, m_out, dummy, idx_v, bufs, gsems, wsems,
               wid, B * Sm // _NW)

    return kb


def kernel(type_table, move_table, cond_table, movetype_idx, moveid_idx,
           condition_idx):
    B, St = movetype_idx.shape
    _, Sm = moveid_idx.shape
    _, Sc = condition_idx.shape

    ka = _make_small_kernel(B, St, Sc, type_table.shape[0],
                            cond_table.shape[0])
    kb = _make_move_kernel(B, Sm)
    t_flat, c_flat = ka(
        type_table, cond_table,
        movetype_idx.T.reshape(-1),
        condition_idx.T.reshape(-1),
    )
    m_flat = kb(move_table, moveid_idx.reshape(-1))
    return (
        t_flat.reshape(St, B, _D).transpose(1, 0, 2),
        m_flat.reshape(B, Sm, _D),
        c_flat.reshape(Sc, B, _D).transpose(1, 0, 2),
    )


# speedup vs baseline: 1.0014x; 1.0014x over previous
import functools

import jax
import jax.numpy as jnp
from jax import lax
from jax.experimental import pallas as pl
from jax.experimental.pallas import tpu as pltpu
from jax.experimental.pallas import tpu_sc as plsc

_D = 128
_NW = 32
_CH = 128
_NBUF = 4

_MESH = plsc.VectorSubcoreMesh(core_axis_name="c", subcore_axis_name="s")


def _phase(table, idx1, out, dummy, idx_v, bufs, gsems, wsems, wid, nrpw):
    nchunks = nrpw // _CH
    base = wid * nrpw

    pltpu.sync_copy(idx1.at[pl.ds(base, nrpw)], idx_v.at[pl.ds(0, nrpw)])

    def gather(c, b):
        pltpu.async_copy(table.at[idx_v.at[pl.ds(c * _CH, _CH)]],
                         bufs[b], gsems[b])

    for b in range(_NBUF - 1):
        gather(b, b)

    def proc(c, b):
        bn = (b + _NBUF - 1) % _NBUF
        @pl.when(c > 0)
        def _():
            pltpu.make_async_copy(bufs[bn], dummy, wsems[bn]).wait()

        @pl.when(c + _NBUF - 1 < nchunks)
        def _():
            gather(c + _NBUF - 1, bn)

        pltpu.make_async_copy(dummy, bufs[b], gsems[b]).wait()
        pltpu.async_copy(bufs[b], out.at[pl.ds(base + c * _CH, _CH)],
                         wsems[b])

    def body(jj, carry):
        for b in range(_NBUF):
            proc(_NBUF * jj + b, b)
        return carry

    lax.fori_loop(0, nchunks // _NBUF, body, 0)
    pltpu.make_async_copy(bufs[_NBUF - 1], dummy, wsems[_NBUF - 1]).wait()


def _make_small_kernel(B, St, Sc, Vt, Vc):
    nmax = B * max(St, Sc) // _NW

    @functools.partial(
        pl.kernel,
        mesh=_MESH,
        out_type=(
            jax.ShapeDtypeStruct((B * St, _D), jnp.float32),
            jax.ShapeDtypeStruct((B * Sc, _D), jnp.float32),
        ),
        scratch_types=(
            [pltpu.VMEM((nmax,), jnp.int32)]
            + [pltpu.VMEM((_CH, _D), jnp.float32)] * _NBUF
            + [pltpu.SemaphoreType.DMA] * (2 * _NBUF)
            + [pltpu.VMEM_SHARED((Vt, _D), jnp.float32),
               pltpu.VMEM_SHARED((Vc, _D), jnp.float32)]
        ),
    )
    def ka(type_table, cond_table, t_idx, c_idx, t_out, c_out, idx_v,
           *rest):
        bufs = list(rest[:_NBUF])
        gsems = list(rest[_NBUF:2 * _NBUF])
        wsems = list(rest[2 * _NBUF:3 * _NBUF])
        type_sh, cond_sh = rest[3 * _NBUF:]
        wid = lax.axis_index("s") * _MESH.num_cores + lax.axis_index("c")
        dummy = type_table.at[pl.ds(0, _CH)]
        args = (dummy, idx_v, bufs, gsems, wsems, wid)

        @pl.when(lax.axis_index("s") == 0)
        def _():
            pltpu.sync_copy(type_table, type_sh)
            pltpu.sync_copy(cond_table, cond_sh)

        plsc.subcore_barrier()
        _phase(type_sh, t_idx, t_out, *args, B * St // _NW)
        _phase(cond_sh, c_idx, c_out, *args, B * Sc // _NW)

    return ka


_BCH = 256
_BNBUF = 3


def _make_move_kernel(B, Sm):
    nrpw = B * Sm // _NW
    nchunks = nrpw // _BCH

    @functools.partial(
        pl.kernel,
        mesh=_MESH,
        out_type=jax.ShapeDtypeStruct((B * Sm, _D), jnp.float32),
        scratch_types=(
            [pltpu.VMEM((nrpw,), jnp.int32)]
            + [pltpu.VMEM((_BCH, _D), jnp.float32)] * _BNBUF
            + [pltpu.SemaphoreType.DMA] * (2 * _BNBUF)
        ),
    )
    def kb(move_table, m_idx, m_out, idx_v, *rest):
        bufs = list(rest[:_BNBUF])
        gsems = list(rest[_BNBUF:2 * _BNBUF])
        wsems = list(rest[2 * _BNBUF:3 * _BNBUF])
        wid = lax.axis_index("s") * _MESH.num_cores + lax.axis_index("c")
        dummy = move_table.at[pl.ds(0, _BCH)]
        base = wid * nrpw

        pltpu.sync_copy(m_idx.at[pl.ds(base, nrpw)],
                        idx_v.at[pl.ds(0, nrpw)])

        def gather(c, b):
            pltpu.async_copy(
                move_table.at[idx_v.at[pl.ds(c * _BCH, _CH)]],
                bufs[b].at[pl.ds(0, _CH)], gsems[b])
            pltpu.async_copy(
                move_table.at[idx_v.at[pl.ds(c * _BCH + _CH, _CH)]],
                bufs[b].at[pl.ds(_CH, _CH)], gsems[b])

        for b in range(_BNBUF - 1):
            gather(b, b)

        def proc(c, b):
            bn = (b + _BNBUF - 1) % _BNBUF
            @pl.when(c > 0)
            def _():
                pltpu.make_async_copy(bufs[bn], dummy, wsems[bn]).wait()

            @pl.when(c + _BNBUF - 1 < nchunks)
            def _():
                gather(c + _BNBUF - 1, bn)

            pltpu.make_async_copy(dummy, bufs[b], gsems[b]).wait()
            pltpu.async_copy(bufs[b], m_out.at[pl.ds(base + c * _BCH, _BCH)],
                             wsems[b])

        def body(jj, carry):
            for b in range(_BNBUF):
                proc(_BNBUF * jj + b, b)
            return carry

        lax.fori_loop(0, nchunks // _BNBUF, body, 0)
        pltpu.make_async_copy(bufs[_BNBUF - 1], dummy,
                              wsems[_BNBUF - 1]).wait()

    return kb


def kernel(type_table, move_table, cond_table, movetype_idx, moveid_idx,
           condition_idx):
    B, St = movetype_idx.shape
    _, Sm = moveid_idx.shape
    _, Sc = condition_idx.shape

    ka = _make_small_kernel(B, St, Sc, type_table.shape[0],
                            cond_table.shape[0])
    kb = _make_move_kernel(B, Sm)
    t_flat, c_flat = ka(
        type_table, cond_table,
        movetype_idx.T.reshape(-1),
        condition_idx.T.reshape(-1),
    )
    m_flat = kb(move_table, moveid_idx.reshape(-1))
    return (
        t_flat.reshape(St, B, _D).transpose(1, 0, 2),
        m_flat.reshape(B, Sm, _D),
        c_flat.reshape(Sc, B, _D).transpose(1, 0, 2),
    )
